# Initial kernel scaffold; baseline (speedup 1.0000x reference)
import jax
import jax.numpy as jnp
from jax.experimental import pallas as pl


def kernel(music_id, genre, audio_features, music_table, genre_table, dense_w, dense_b):
    raise NotImplementedError("write your pallas kernel here")


# speedup vs baseline: 1.6922x; 1.6922x over previous
import functools

import jax
import jax.numpy as jnp
from jax import lax
from jax.experimental import pallas as pl
from jax.experimental.pallas import tpu as pltpu
from jax.experimental.pallas import tpu_sc as plsc

NUM_MUSIC = 1000000
NUM_GENRES = 1000
DIM_MUSIC = 128
DIM_GENRE = 16
DIM_AUDIO = 32
BATCH = 16384
DIM_OUT = DIM_MUSIC + DIM_GENRE + DIM_AUDIO

_NC = 2
_NS = 16
_NW = _NC * _NS
_BPW = BATCH // _NW

_mesh = plsc.VectorSubcoreMesh(core_axis_name="c", subcore_axis_name="s")


_CH = 128
_NCHUNK = _BPW // _CH
_DIM_TAIL = DIM_GENRE + DIM_AUDIO


@functools.partial(
    pl.kernel,
    mesh=_mesh,
    out_type=jax.ShapeDtypeStruct((BATCH, DIM_OUT), jnp.float32),
    scratch_types=[
        pltpu.VMEM((_CH,), jnp.int32),
        pltpu.VMEM((_CH,), jnp.int32),
        pltpu.VMEM((_CH, DIM_MUSIC), jnp.float32),
        pltpu.VMEM((_CH, DIM_AUDIO), jnp.float32),
        pltpu.VMEM((_CH, _DIM_TAIL), jnp.float32),
        pltpu.VMEM((DIM_GENRE, NUM_GENRES), jnp.float32),
        pltpu.SemaphoreType.DMA,
    ],
    compiler_params=pltpu.CompilerParams(needs_layout_passes=False),
)
def _sc_assemble(music_id_hbm, genre_hbm, aproj_hbm, music_tab_hbm,
                 genre_tabT_hbm, out_hbm, idx_m, idx_g, buf_m, buf_a,
                 buf_t, gtab_v, sem_m):
    wid = lax.axis_index("s") * _NC + lax.axis_index("c")
    base = wid * _BPW
    pltpu.sync_copy(genre_tabT_hbm, gtab_v)

    for k in range(_NCHUNK):
        b0 = base + k * _CH
        pltpu.sync_copy(music_id_hbm.at[pl.ds(b0, _CH)], idx_m)
        pltpu.sync_copy(genre_hbm.at[pl.ds(b0, _CH)], idx_g)
        cm = pltpu.async_copy(music_tab_hbm.at[idx_m], buf_m, sem_m)
        pltpu.sync_copy(aproj_hbm.at[pl.ds(b0, _CH)], buf_a)

        lanes = lax.iota(jnp.int32, 16)
        for i8 in range(_CH // 16):
            gv = idx_g[pl.ds(i8 * 16, 16)]
            rows = lanes + i8 * 16
            for j in range(DIM_GENRE):
                vals = plsc.load_gather(gtab_v, [jnp.full((16,), j, jnp.int32), gv])
                plsc.store_scatter(buf_t, [rows, jnp.full((16,), j, jnp.int32)], vals)

        def _asm(i, carry):
            buf_t[i, pl.ds(16, 16)] = buf_a[i, pl.ds(0, 16)]
            buf_t[i, pl.ds(32, 16)] = buf_a[i, pl.ds(16, 16)]
            return carry

        lax.fori_loop(0, _CH, _asm, 0)
        cm.wait()
        pltpu.sync_copy(buf_m, out_hbm.at[pl.ds(b0, _CH), pl.ds(0, DIM_MUSIC)])
        pltpu.sync_copy(buf_t,
                        out_hbm.at[pl.ds(b0, _CH), pl.ds(DIM_MUSIC, _DIM_TAIL)])


def _mm_body(a_ref, w_ref, b_ref, o_ref):
    o_ref[...] = jnp.dot(a_ref[...], w_ref[...],
                         preferred_element_type=jnp.float32) + b_ref[...]


_BM = 2048


def _audio_proj(audio, w, b):
    return pl.pallas_call(
        _mm_body,
        grid=(BATCH // _BM,),
        in_specs=[
            pl.BlockSpec((_BM, DIM_AUDIO), lambda i: (i, 0)),
            pl.BlockSpec((DIM_AUDIO, DIM_AUDIO), lambda i: (0, 0)),
            pl.BlockSpec((1, DIM_AUDIO), lambda i: (0, 0)),
        ],
        out_specs=pl.BlockSpec((_BM, DIM_AUDIO), lambda i: (i, 0)),
        out_shape=jax.ShapeDtypeStruct((BATCH, DIM_AUDIO), jnp.float32),
    )(audio, w, b.reshape(1, DIM_AUDIO))


def kernel(music_id, genre, audio_features, music_table, genre_table,
           dense_w, dense_b):
    aproj = _audio_proj(audio_features, dense_w, dense_b)
    return _sc_assemble(
        jnp.asarray(music_id, jnp.int32),
        jnp.asarray(genre, jnp.int32),
        aproj,
        music_table,
        genre_table.T,
    )


# speedup vs baseline: 3.3179x; 1.9607x over previous
import functools

import jax
import jax.numpy as jnp
from jax import lax
from jax.experimental import pallas as pl
from jax.experimental.pallas import tpu as pltpu
from jax.experimental.pallas import tpu_sc as plsc

NUM_MUSIC = 1000000
NUM_GENRES = 1000
DIM_MUSIC = 128
DIM_GENRE = 16
DIM_AUDIO = 32
BATCH = 16384
DIM_OUT = DIM_MUSIC + DIM_GENRE + DIM_AUDIO

_NC = 2
_NS = 16
_NW = _NC * _NS
_BPW = BATCH // _NW
_CH = 128
_NCHUNK = _BPW // _CH

_mesh = plsc.VectorSubcoreMesh(core_axis_name="c", subcore_axis_name="s")


@functools.partial(
    pl.kernel,
    mesh=_mesh,
    out_type=(
        jax.ShapeDtypeStruct((BATCH, DIM_MUSIC), jnp.float32),
        jax.ShapeDtypeStruct((DIM_GENRE, BATCH), jnp.float32),
    ),
    scratch_types=[
        pltpu.VMEM((_BPW,), jnp.int32),
        pltpu.VMEM((_BPW,), jnp.int32),
        pltpu.VMEM((_CH, DIM_MUSIC), jnp.float32),
        pltpu.VMEM((_CH, DIM_MUSIC), jnp.float32),
        pltpu.VMEM((_CH, DIM_MUSIC), jnp.float32),
        pltpu.VMEM((_CH, DIM_MUSIC), jnp.float32),
        pltpu.VMEM((DIM_GENRE, _BPW), jnp.float32),
        pltpu.VMEM((DIM_GENRE, NUM_GENRES), jnp.float32),
        pltpu.SemaphoreType.DMA,
        pltpu.SemaphoreType.DMA,
        pltpu.SemaphoreType.DMA,
        pltpu.SemaphoreType.DMA,
        pltpu.SemaphoreType.DMA,
        pltpu.SemaphoreType.DMA,
    ],
    compiler_params=pltpu.CompilerParams(needs_layout_passes=False),
)
def _sc_gather(music_id_hbm, genre_hbm, music_tab_hbm, genre_tabT_hbm,
               m_hbm, gT_hbm, idx_m, idx_g, buf_m0, buf_m1, buf_m2, buf_m3,
               buf_gT, gtab_v, sem0, sem1, sem2, sem3, semw0, semw1):
    wid = lax.axis_index("s") * _NC + lax.axis_index("c")
    base = wid * _BPW
    pltpu.sync_copy(music_id_hbm.at[pl.ds(base, _BPW)], idx_m)
    bufs = (buf_m0, buf_m1, buf_m2, buf_m3)
    sems = (sem0, sem1, sem2, sem3)
    copies = [
        pltpu.async_copy(music_tab_hbm.at[idx_m.at[pl.ds(k * _CH, _CH)]],
                         bufs[k], sems[k])
        for k in range(_NCHUNK)
    ]
    pltpu.sync_copy(genre_tabT_hbm, gtab_v)
    pltpu.sync_copy(genre_hbm.at[pl.ds(base, _BPW)], idx_g)

    def _genre_step(i8, carry):
        gv = idx_g[pl.ds(i8 * 16, 16)]
        for j in range(DIM_GENRE):
            buf_gT[j, pl.ds(i8 * 16, 16)] = plsc.load_gather(
                gtab_v, [jnp.full((16,), j, jnp.int32), gv])
        return carry

    lax.fori_loop(0, _BPW // 16, _genre_step, 0)

    wsems = (semw0, semw1)
    wcopies = [None, None]
    for k in range(_NCHUNK):
        copies[k].wait()
        if wcopies[k % 2] is not None:
            wcopies[k % 2].wait()
        wcopies[k % 2] = pltpu.async_copy(
            bufs[k], m_hbm.at[pl.ds(base + k * _CH, _CH)], wsems[k % 2])
    pltpu.sync_copy(buf_gT, gT_hbm.at[:, pl.ds(base, _BPW)])
    for w in wcopies:
        if w is not None:
            w.wait()


_BM = 8192


def _tc_body(m_ref, gT_ref, aT_ref, w_ref, b_ref, o_ref):
    o_ref[0:DIM_MUSIC, :] = m_ref[...].T
    o_ref[DIM_MUSIC:DIM_MUSIC + DIM_GENRE, :] = gT_ref[...]
    ap_t = lax.dot_general(w_ref[...], aT_ref[...], (((0,), (0,)), ((), ())),
                           preferred_element_type=jnp.float32)
    o_ref[DIM_MUSIC + DIM_GENRE:DIM_OUT, :] = ap_t + b_ref[...]


def _tc_assemble(m, gembT, audioT, w, b2):
    return pl.pallas_call(
        _tc_body,
        grid=(BATCH // _BM,),
        in_specs=[
            pl.BlockSpec((_BM, DIM_MUSIC), lambda i: (i, 0)),
            pl.BlockSpec((DIM_GENRE, _BM), lambda i: (0, i)),
            pl.BlockSpec((DIM_AUDIO, _BM), lambda i: (0, i)),
            pl.BlockSpec((DIM_AUDIO, DIM_AUDIO), lambda i: (0, 0)),
            pl.BlockSpec((DIM_AUDIO, 1), lambda i: (0, 0)),
        ],
        out_specs=pl.BlockSpec((DIM_OUT, _BM), lambda i: (0, i)),
        out_shape=jax.ShapeDtypeStruct((DIM_OUT, BATCH), jnp.float32),
    )(m, gembT, audioT, w, b2)


def kernel(music_id, genre, audio_features, music_table, genre_table,
           dense_w, dense_b):
    m, gembT = _sc_gather(
        jnp.asarray(music_id, jnp.int32),
        jnp.asarray(genre, jnp.int32),
        music_table,
        genre_table.T,
    )
    out_t = _tc_assemble(m, gembT, audio_features.T, dense_w,
                         dense_b.reshape(DIM_AUDIO, 1))
    return out_t.T
